# baseline (speedup 1.0000x reference)
import jax
import jax.numpy as jnp
import numpy as np
from jax import lax
from jax.experimental import pallas as pl
from jax.experimental.pallas import tpu as pltpu
from jax.experimental.pallas import tpu_sc as plsc

NG, NN, NFEAT = 256, 512, 9
HIDDEN = 128
NC, NS = 2, 16
NW = NC * NS
UNITS = 2 * NG
UPW = UNITS // NW
NODES_U = NN // 2
AROWS = NODES_U * NFEAT // HIDDEN
DROWS = NODES_U // HIDDEN
OUT_ROWS = NG * (NN + 1)
NLANE = HIDDEN // 16


def _body(nf2, di2, dout2, ae, de, doe, gt, out,
          idx_v, dio_v, gb0, gb1, gb2, gt_v, accs,
          gs0, gs1, gs2, isem, osem):
    cid = lax.axis_index("c")
    sid = lax.axis_index("s")
    wid = cid * NS + sid

    pltpu.sync_copy(gt, gt_v)

    def _prefetch(u, pb):
        pltpu.async_copy(nf2.at[pl.ds(u * AROWS, AROWS)], idx_v.at[pb], isem)
        pltpu.async_copy(di2.at[pl.ds(u * DROWS, DROWS)],
                         dio_v.at[pb, pl.ds(0, DROWS)], isem)
        pltpu.async_copy(dout2.at[pl.ds(u * DROWS, DROWS)],
                         dio_v.at[pb, pl.ds(DROWS, DROWS)], isem)

    def _wait_prefetch(pb):
        pltpu.make_async_copy(nf2.at[pl.ds(0, AROWS)], idx_v.at[pb], isem).wait()
        pltpu.make_async_copy(di2.at[pl.ds(0, DROWS)],
                              dio_v.at[pb, pl.ds(0, DROWS)], isem).wait()
        pltpu.make_async_copy(dout2.at[pl.ds(0, DROWS)],
                              dio_v.at[pb, pl.ds(DROWS, DROWS)], isem).wait()

    def _wait64k(sem, gb):
        pltpu.make_async_copy(ae.at[pl.ds(0, HIDDEN)], gb, sem).wait()

    def _acc_rows_dyn(gb, hb, n, jlo, jhi):
        @pl.when(jhi > jlo)
        def _():
            def rb(j, accv):
                return tuple(accv[l] + gb[j, pl.ds(l * 16, 16)]
                             for l in range(NLANE))
            z = tuple(jnp.zeros((16,), jnp.float32) for _ in range(NLANE))
            accv = lax.fori_loop(jlo, jhi, rb, z)
            for l in range(NLANE):
                plsc.addupdate(accs.at[hb, n, pl.ds(l * 16, 16)], accv[l])

    def _process_atom(gb, hb, r):
        jbase = 128 * r
        n0 = jbase // 9
        hoff = jbase - 9 * n0
        nlo = n0 + jnp.where(hoff > 0, 1, 0)
        j0f = 9 * nlo - jbase
        nfull = (128 - j0f) // 9
        _acc_rows_dyn(gb, hb, n0, jnp.int32(0), j0f)

        def fn(i, _):
            n = nlo + i
            j0 = j0f + 9 * i
            for l in range(NLANE):
                sl = pl.ds(l * 16, 16)
                v = gb[j0, sl]
                for o in range(1, NFEAT):
                    v = v + gb[j0 + o, sl]
                plsc.addupdate(accs.at[hb, n, sl], v)
            return 0

        lax.fori_loop(0, nfull, fn, 0)
        jt = j0f + 9 * nfull
        _acc_rows_dyn(gb, hb, nlo + nfull, jt, jnp.int32(128))

    def _process_doe(gb, hb, noff):
        def dr(j, _):
            for l in range(NLANE):
                sl = pl.ds(l * 16, 16)
                plsc.addupdate(accs.at[hb, noff + j, sl], gb[j, sl])
            return 0
        lax.fori_loop(0, HIDDEN, dr, 0)

    _prefetch(wid * UPW, 0)

    def _unit(ui, _):
        u = wid * UPW + ui
        g = u // 2
        h = u - g * 2
        pb = ui % 2
        hb = ui % 2
        _wait_prefetch(pb)

        @pl.when(ui + 1 < UPW)
        def _():
            _prefetch(u + 1, 1 - pb)

        pltpu.async_copy(de.at[dio_v.at[pb, 0]],
                         accs.at[hb, pl.ds(0, HIDDEN)], gs0)
        pltpu.async_copy(de.at[dio_v.at[pb, 1]],
                         accs.at[hb, pl.ds(HIDDEN, HIDDEN)], gs1)
        pltpu.async_copy(doe.at[dio_v.at[pb, 2]], gb0, gs0)
        pltpu.async_copy(doe.at[dio_v.at[pb, 3]], gb1, gs1)
        pltpu.async_copy(ae.at[idx_v.at[pb, 0]], gb2, gs2)

        _wait64k(gs0, gb0)
        _wait64k(gs0, gb0)
        _process_doe(gb0, hb, 0)
        pltpu.async_copy(ae.at[idx_v.at[pb, 1]], gb0, gs0)
        _wait64k(gs1, gb1)
        _wait64k(gs1, gb1)
        _process_doe(gb1, hb, HIDDEN)
        pltpu.async_copy(ae.at[idx_v.at[pb, 2]], gb1, gs1)

        rot = ((gb2, gs2), (gb0, gs0), (gb1, gs1))

        def _aloop(t, _):
            for i, (gb, gs) in enumerate(rot):
                r = 3 * t + i
                _wait64k(gs, gb)
                _process_atom(gb, hb, r)
                pltpu.async_copy(ae.at[idx_v.at[pb, r + 3]], gb, gs)
            return 0

        lax.fori_loop(0, AROWS // 3 - 1, _aloop, 0)

        for i, (gb, gs) in enumerate(rot):
            r = AROWS - 3 + i
            _wait64k(gs, gb)
            _process_atom(gb, hb, jnp.int32(r))

        @pl.when(ui > 0)
        def _():
            pltpu.make_async_copy(ae.at[pl.ds(0, NODES_U)],
                                  accs.at[hb], osem).wait()

        pltpu.async_copy(accs.at[hb],
                         out.at[pl.ds(g * (NN + 1) + 1 + h * NODES_U, NODES_U)],
                         osem)

        @pl.when(h == 0)
        def _():
            pltpu.sync_copy(gt_v, out.at[pl.ds(g * (NN + 1), 1)])
        return 0

    lax.fori_loop(0, UPW, _unit, 0)
    pltpu.make_async_copy(ae.at[pl.ds(0, NODES_U)],
                          accs.at[(UPW - 1) % 2], osem).wait()


_sc_call = pl.kernel(
    _body,
    out_type=jax.ShapeDtypeStruct((OUT_ROWS, HIDDEN), jnp.float32),
    mesh=plsc.VectorSubcoreMesh(core_axis_name="c", subcore_axis_name="s",
                                num_cores=NC, num_subcores=NS),
    scratch_types=[
        pltpu.VMEM((2, AROWS, HIDDEN), jnp.int32),
        pltpu.VMEM((2, 2 * DROWS, HIDDEN), jnp.int32),
        pltpu.VMEM((HIDDEN, HIDDEN), jnp.float32),
        pltpu.VMEM((HIDDEN, HIDDEN), jnp.float32),
        pltpu.VMEM((HIDDEN, HIDDEN), jnp.float32),
        pltpu.VMEM((1, HIDDEN), jnp.float32),
        pltpu.VMEM((2, NODES_U, HIDDEN), jnp.float32),
        pltpu.SemaphoreType.DMA,
        pltpu.SemaphoreType.DMA,
        pltpu.SemaphoreType.DMA,
        pltpu.SemaphoreType.DMA,
        pltpu.SemaphoreType.DMA,
    ],
    compiler_params=pltpu.CompilerParams(use_tc_tiling_on_sc=False),
)


@jax.jit
def kernel(nf, di, dout, ae, de, doe, gt):
    nf2 = nf.astype(jnp.int32).reshape(UNITS * AROWS, HIDDEN)
    di2 = di.astype(jnp.int32).reshape(UNITS * DROWS, HIDDEN)
    dout2 = dout.astype(jnp.int32).reshape(UNITS * DROWS, HIDDEN)
    out = _sc_call(nf2, di2, dout2, ae, de, doe, gt)
    return out.reshape(NG, NN + 1, HIDDEN)
